# Initial kernel scaffold; baseline (speedup 1.0000x reference)
import jax
import jax.numpy as jnp
from jax.experimental import pallas as pl


def kernel(inputs, topic_vectors):
    raise NotImplementedError("write your pallas kernel here")


# speedup vs baseline: 10.8590x; 10.8590x over previous
import jax
import jax.numpy as jnp
from jax import lax
from jax.experimental import pallas as pl
from jax.experimental.pallas import tpu as pltpu

NUM_TOPICS = 512
TOP_K = 8
DIM = 1024


def _body(tvT_ref, out_ref, sheet_ref):
    b = pl.program_id(0)
    h = pl.program_id(1)

    @pl.when((b == 0) & (h == 0))
    def _compute():
        x = tvT_ref[...]
        iota = lax.broadcasted_iota(jnp.int32, x.shape, 0)
        acc = jnp.zeros(x.shape, jnp.float32)
        neg_inf = jnp.float32(-jnp.inf)
        for _ in range(TOP_K):
            m = jnp.max(x, axis=0, keepdims=True)
            cand = jnp.where(x == m, iota, jnp.int32(DIM))
            amin = jnp.min(cand, axis=0, keepdims=True)
            onehot = iota == amin
            acc = jnp.where(onehot, jnp.float32(1.0), acc)
            x = jnp.where(onehot, neg_inf, x)
        sheet_ref[...] = acc

    @pl.when(h == 0)
    def _copy_sheet():
        out_ref[0, :, :] = sheet_ref[...]

    @pl.when(h == 1)
    def _zero_tail():
        out_ref[0, :, :] = jnp.zeros((out_ref.shape[1], NUM_TOPICS), jnp.float32)


def kernel(inputs, topic_vectors):
    _, batch, max_len, _ = inputs.shape
    tvT = topic_vectors.T
    half = max_len // 2
    out = pl.pallas_call(
        _body,
        grid=(batch, 2),
        in_specs=[pl.BlockSpec((DIM, NUM_TOPICS), lambda b, h: (0, 0))],
        out_specs=pl.BlockSpec((1, half, NUM_TOPICS), lambda b, h: (b, h, 0)),
        out_shape=jax.ShapeDtypeStruct((batch, max_len, NUM_TOPICS), jnp.float32),
        scratch_shapes=[pltpu.VMEM((DIM, NUM_TOPICS), jnp.float32)],
    )(tvT)
    return out


# speedup vs baseline: 14.0606x; 1.2948x over previous
import jax
import jax.numpy as jnp
from jax import lax
from jax.experimental import pallas as pl
from jax.experimental.pallas import tpu as pltpu

NUM_TOPICS = 512
TOP_K = 8
DIM = 1024


def _body(tvT_ref, out_hbm, zbuf, sheet, sems):
    batch = out_hbm.shape[0]
    max_len = out_hbm.shape[1]

    zbuf[...] = jnp.zeros(zbuf.shape, jnp.float32)
    zcopies = []
    for b in range(batch):
        c = pltpu.make_async_copy(
            zbuf, out_hbm.at[b, pl.ds(DIM, max_len - DIM), :], sems.at[b]
        )
        c.start()
        zcopies.append(c)

    x0 = tvT_ref[...]
    iota = lax.broadcasted_iota(jnp.int32, x0.shape, 0)
    neg_inf = jnp.float32(-jnp.inf)
    x = x0
    for _ in range(TOP_K):
        m = jnp.max(x, axis=0, keepdims=True)
        cand = jnp.where(x == m, iota, jnp.int32(DIM))
        amin = jnp.min(cand, axis=0, keepdims=True)
        x = jnp.where(iota == amin, neg_inf, x)
    sheet[...] = jnp.where(x != x0, jnp.float32(1.0), jnp.float32(0.0))

    scopies = []
    for b in range(batch):
        c = pltpu.make_async_copy(
            sheet, out_hbm.at[b, pl.ds(0, DIM), :], sems.at[batch + b]
        )
        c.start()
        scopies.append(c)

    for c in zcopies + scopies:
        c.wait()


def kernel(inputs, topic_vectors):
    _, batch, max_len, _ = inputs.shape
    tvT = topic_vectors.T
    out = pl.pallas_call(
        _body,
        in_specs=[pl.BlockSpec(memory_space=pltpu.MemorySpace.VMEM)],
        out_specs=pl.BlockSpec(memory_space=pltpu.MemorySpace.HBM),
        out_shape=jax.ShapeDtypeStruct((batch, max_len, NUM_TOPICS), jnp.float32),
        scratch_shapes=[
            pltpu.VMEM((max_len - DIM, NUM_TOPICS), jnp.float32),
            pltpu.VMEM((DIM, NUM_TOPICS), jnp.float32),
            pltpu.SemaphoreType.DMA((2 * batch,)),
        ],
    )(tvT)
    return out
